# baseline (speedup 1.0000x reference)
import functools
import math

import jax
import jax.numpy as jnp
from jax import lax
from jax.experimental import pallas as pl
from jax.experimental.pallas import tpu as pltpu
from jax.experimental.pallas import tpu_sc as plsc

_B, _S, _V = 64, 4, 100000
_LS = 0.1
_CONF = 1.0 - _LS
_SV = _LS / (_V - 2)
_ENT = (_V - 2) * _SV * math.log(_SV) + _CONF * math.log(_CONF)

_TCB = 48
_VB = 8192
_NBLK = (_V + _VB - 1) // _VB


def _tc_kernel(t_ref, x_ref, o_ref, acc_ref, tb_ref):
    j = pl.program_id(0)
    lane = jax.lax.broadcasted_iota(jnp.int32, (_TCB, _S, _VB), 2)

    @pl.when(j == 0)
    def _():
        t = t_ref[...]
        tb_ref[...] = jnp.broadcast_to(t, (_TCB, _S, _VB))
        x = x_ref[...]
        sel = jnp.where(lane == tb_ref[...], -_CONF, -_SV)
        sel = jnp.where(lane == 0, 0.0, sel)
        acc_ref[...] = x * sel

    @pl.when((j > 0) & (j < _NBLK - 1))
    def _():
        x = x_ref[...]
        gl = lane + j * _VB
        sel = jnp.where(gl == tb_ref[...], -_CONF, -_SV)
        acc_ref[...] = acc_ref[...] + x * sel

    @pl.when(j == _NBLK - 1)
    def _():
        x = x_ref[...]
        gl = lane + j * _VB
        sel = jnp.where(gl == tb_ref[...], -_CONF, -_SV)
        sel = jnp.where(gl >= _V, 0.0, sel)
        acc_ref[...] = acc_ref[...] + jnp.where(gl >= _V, 0.0, x * sel)
        t = t_ref[...]
        wrow = jnp.where(t == 0, 0.0, 1.0)
        rowvals = jnp.sum(acc_ref[...], axis=2, keepdims=True)
        contrib = wrow * (jnp.float32(_ENT) + rowvals)
        o_ref[0, 0] = jnp.sum(contrib)


def _tc_partial(t3, output):
    out = pl.pallas_call(
        _tc_kernel,
        grid=(_NBLK,),
        in_specs=[
            pl.BlockSpec((_TCB, _S, 1), lambda j: (0, 0, 0)),
            pl.BlockSpec((_TCB, _S, _VB), lambda j: (0, 0, j)),
        ],
        out_specs=pl.BlockSpec(memory_space=pltpu.SMEM),
        out_shape=jax.ShapeDtypeStruct((1, 1), jnp.float32),
        scratch_shapes=[
            pltpu.VMEM((_TCB, _S, _VB), jnp.float32),
            pltpu.VMEM((_TCB, _S, _VB), jnp.int32),
        ],
        compiler_params=pltpu.CompilerParams(
            dimension_semantics=("arbitrary",),
        ),
    )(t3, output)
    return out[0, 0]


_SCB0 = _TCB
_CH = 2048
_NCH = 24
_HALFV = _NCH * _CH
_TAIL = _V - 2 * _HALFV


def _sc_partials(output, target):
    info = plsc.get_sparse_core_info()
    nc = info.num_cores
    mesh = plsc.VectorSubcoreMesh(core_axis_name="c", subcore_axis_name="s")

    @functools.partial(
        pl.kernel,
        mesh=mesh,
        out_type=jax.ShapeDtypeStruct((32, 16), jnp.float32),
        scratch_types=[
            pltpu.VMEM((_S, 16), jnp.int32),
            pltpu.VMEM((_S, _CH), jnp.float32),
            pltpu.VMEM((_S, _CH), jnp.float32),
            pltpu.VMEM((_S, _TAIL), jnp.float32),
            pltpu.VMEM((16,), jnp.float32),
            pltpu.SemaphoreType.DMA,
            pltpu.SemaphoreType.DMA,
            pltpu.SemaphoreType.DMA,
        ],
    )
    def k(x_hbm, t_hbm, out_hbm, tv, buf0, buf1, tbuf, accv, sem0, sem1, semt):
        w = lax.axis_index("s") * nc + lax.axis_index("c")
        b = _SCB0 + w // 2
        half = w % 2
        pltpu.sync_copy(t_hbm.at[b], tv)
        iota = lax.iota(jnp.int32, 16)
        one = jnp.full((16,), 1, jnp.int32)
        tsp, csv, cdel = [], [], []
        for si in range(_S):
            t_s = tv[si, :]
            wm = jnp.minimum(jnp.abs(t_s), one).astype(jnp.float32)
            tsp.append(t_s)
            csv.append(wm * jnp.float32(-_SV))
            cdel.append(wm * jnp.float32(_SV - _CONF))

        base = half * _HALFV
        bufs = [buf0, buf1]
        sems = [sem0, sem1]

        def chunk_loop(buf, off, nvec, acc):
            tso = [t - off for t in tsp]
            zso = jnp.broadcast_to(-off, (16,))

            def body(i, carry):
                acc, g = carry
                znot = jnp.minimum(jnp.abs(g - zso), one).astype(jnp.float32)
                for si in range(_S):
                    v = buf[si, pl.ds(i * 16, 16)]
                    meq = (one - jnp.minimum(jnp.abs(g - tso[si]), one)
                           ).astype(jnp.float32)
                    sel = (csv[si] + cdel[si] * meq) * znot
                    acc = acc + v * sel
                return acc, g + 16
            acc, _ = lax.fori_loop(0, nvec, body, (acc, iota))
            return acc

        acc = jnp.zeros((16,), jnp.float32)
        cps = {0: pltpu.async_copy(
            x_hbm.at[b, :, pl.ds(base, _CH)], buf0, sem0)}
        for kk in range(_NCH):
            cps[kk].wait()
            if kk + 1 < _NCH:
                cps[kk + 1] = pltpu.async_copy(
                    x_hbm.at[b, :, pl.ds(base + (kk + 1) * _CH, _CH)],
                    bufs[(kk + 1) % 2], sems[(kk + 1) % 2])
            acc = chunk_loop(bufs[kk % 2], base + kk * _CH, _CH // 16, acc)

        pltpu.sync_copy(x_hbm.at[b, :, pl.ds(2 * _HALFV, _TAIL)], tbuf)
        tail = chunk_loop(tbuf, jnp.int32(2 * _HALFV), _TAIL // 16,
                          jnp.zeros((16,), jnp.float32))
        accv[...] = acc + tail * half.astype(jnp.float32)
        pltpu.sync_copy(accv, out_hbm.at[w])

    tmat = jnp.broadcast_to(target.reshape(_B, _S, 1), (_B, _S, 16))
    return k(output, tmat)


def kernel(output, target, one_hot):
    del one_hot
    t3 = target.reshape(_B, _S, 1)
    tc = _tc_partial(t3, output)
    sc = _sc_partials(output, target)
    n_sc = jnp.sum((target[_SCB0:, :] != 0).astype(jnp.float32))
    return tc + jnp.sum(sc) + jnp.float32(_ENT) * n_sc
